# baseline (speedup 1.0000x reference)
import functools

import jax
import jax.numpy as jnp
from jax import lax
from jax.experimental import pallas as pl
from jax.experimental.pallas import tpu as pltpu
from jax.experimental.pallas import tpu_sc as plsc

CLIP = 64


def _build_band_kernel(table_ref, e_ref, *, S, C, D):
    e_ref[0 : S - C, :] = jnp.broadcast_to(table_ref[0:1, :], (S - C, D))
    e_ref[S - C : S - 1 + C, :] = table_ref[1 : 2 * C, :]
    e_ref[S - 1 + C :, :] = jnp.broadcast_to(table_ref[2 * C : 2 * C + 1, :], (S - C + 1, D))


def _make_sc_window_kernel(S, D, NC, NS):
    n_rows = S // (NC * NS)
    mesh = plsc.VectorSubcoreMesh(core_axis_name="c", subcore_axis_name="s")

    @functools.partial(
        pl.kernel,
        out_type=jax.ShapeDtypeStruct((S, S, D), jnp.float32),
        mesh=mesh,
        scratch_types=[
            pltpu.VMEM_SHARED((2 * S, D), jnp.float32),
            pltpu.SemaphoreType.DMA,
            pltpu.SemaphoreType.DMA,
        ],
    )
    def sc_kernel(e_hbm, out_hbm, e_sh, sem_in, sem):
        cid = lax.axis_index("c")
        sid = lax.axis_index("s")

        @pl.when(sid == 0)
        def _stage_band():
            pltpu.make_async_copy(e_hbm, e_sh, sem_in).start()
            pltpu.make_async_copy(e_hbm, e_sh, sem_in).wait()

        plsc.subcore_barrier()

        wid = sid * NC + cid
        base = wid * n_rows

        def body(t, _):
            r = base + t
            cp = pltpu.make_async_copy(
                e_sh.at[pl.ds(S - 1 - r, S), :], out_hbm.at[r], sem
            )
            cp.start()
            cp.wait()
            return 0

        lax.fori_loop(0, n_rows, body, 0)

    return sc_kernel


def _rel_pos_encoding(table, S, C, D, interpret=False):
    band = pl.pallas_call(
        lambda t, e: _build_band_kernel(t, e, S=S, C=C, D=D),
        in_specs=[pl.BlockSpec(memory_space=pltpu.VMEM)],
        out_specs=pl.BlockSpec(memory_space=pltpu.VMEM),
        out_shape=jax.ShapeDtypeStruct((2 * S, D), table.dtype),
        interpret=interpret,
    )(table)
    sc_kernel = _make_sc_window_kernel(S, D, 2, 16)
    return sc_kernel(band)


def kernel(x, encoding_matrix):
    S = x.shape[1]
    D = encoding_matrix.shape[1]
    return _rel_pos_encoding(encoding_matrix, S, CLIP, D)


# speedup vs baseline: 1.0083x; 1.0083x over previous
import functools

import jax
import jax.numpy as jnp
from jax import lax
from jax.experimental import pallas as pl
from jax.experimental.pallas import tpu as pltpu
from jax.experimental.pallas import tpu_sc as plsc

CLIP = 64


def _build_band_kernel(table_ref, e_ref, *, S, C, D):
    e_ref[0 : S - C, :] = jnp.broadcast_to(table_ref[0:1, :], (S - C, D))
    e_ref[S - C : S - 1 + C, :] = table_ref[1 : 2 * C, :]
    e_ref[S - 1 + C :, :] = jnp.broadcast_to(table_ref[2 * C : 2 * C + 1, :], (S - C + 1, D))


def _make_sc_window_kernel(S, D, NC, NS):
    n_rows = S // (NC * NS)
    mesh = plsc.VectorSubcoreMesh(core_axis_name="c", subcore_axis_name="s")

    @functools.partial(
        pl.kernel,
        out_type=jax.ShapeDtypeStruct((S, S, D), jnp.float32),
        mesh=mesh,
        scratch_types=[
            pltpu.VMEM_SHARED((2 * S, D), jnp.float32),
            pltpu.SemaphoreType.DMA,
            pltpu.SemaphoreType.DMA,
        ],
    )
    def sc_kernel(e_hbm, out_hbm, e_sh, sem_in, sem):
        cid = lax.axis_index("c")
        sid = lax.axis_index("s")

        @pl.when(sid == 0)
        def _stage_band():
            pltpu.make_async_copy(e_hbm, e_sh, sem_in).start()
            pltpu.make_async_copy(e_hbm, e_sh, sem_in).wait()

        plsc.subcore_barrier()

        wid = sid * NC + cid
        base = wid * n_rows
        K = 8

        def mk(t):
            r = base + t
            return pltpu.make_async_copy(
                e_sh.at[pl.ds(S - 1 - r, S), :], out_hbm.at[r], sem
            )

        def body(t, _):
            @pl.when(t >= K)
            def _():
                mk(t - K).wait()

            mk(t).start()
            return 0

        lax.fori_loop(0, n_rows, body, 0)

        def drain(k, _):
            mk(n_rows - K + k).wait()
            return 0

        lax.fori_loop(0, K, drain, 0)

    return sc_kernel


def _rel_pos_encoding(table, S, C, D, interpret=False):
    band = pl.pallas_call(
        lambda t, e: _build_band_kernel(t, e, S=S, C=C, D=D),
        in_specs=[pl.BlockSpec(memory_space=pltpu.VMEM)],
        out_specs=pl.BlockSpec(memory_space=pltpu.VMEM),
        out_shape=jax.ShapeDtypeStruct((2 * S, D), table.dtype),
        interpret=interpret,
    )(table)
    sc_kernel = _make_sc_window_kernel(S, D, 2, 16)
    return sc_kernel(band)


def kernel(x, encoding_matrix):
    S = x.shape[1]
    D = encoding_matrix.shape[1]
    return _rel_pos_encoding(encoding_matrix, S, CLIP, D)
